# Initial kernel scaffold; baseline (speedup 1.0000x reference)
import jax
import jax.numpy as jnp
from jax.experimental import pallas as pl


def kernel(node_features, edge_index, params):
    raise NotImplementedError("write your pallas kernel here")


# speedup vs baseline: 1.0272x; 1.0272x over previous
import jax
import jax.numpy as jnp
from jax.experimental import pallas as pl
from jax.experimental.pallas import tpu as pltpu

N = 10000
E = 320000
DF = 128
H = 128
L = 4
HEADS = 4
HD = H // HEADS

NB = 400
NPAD = 10240


def _encoder_body(nf_ref, w1_ref, b1_ref, w2_ref, b2_ref, g_ref, bb_ref, o_ref):
    x = jnp.maximum(jnp.dot(nf_ref[...], w1_ref[...],
                            preferred_element_type=jnp.float32) + b1_ref[...], 0.0)
    x = jnp.dot(x, w2_ref[...], preferred_element_type=jnp.float32) + b2_ref[...]
    m = x.mean(-1, keepdims=True)
    v = ((x - m) ** 2).mean(-1, keepdims=True)
    o_ref[...] = (x - m) * jax.lax.rsqrt(v + 1e-5) * g_ref[...] + bb_ref[...]


def _encoder(node_features, p):
    grid = (N // NB,)
    return pl.pallas_call(
        _encoder_body,
        grid=grid,
        in_specs=[
            pl.BlockSpec((NB, DF), lambda i: (i, 0)),
            pl.BlockSpec((DF, H), lambda i: (0, 0)),
            pl.BlockSpec((H,), lambda i: (0,)),
            pl.BlockSpec((H, H), lambda i: (0, 0)),
            pl.BlockSpec((H,), lambda i: (0,)),
            pl.BlockSpec((H,), lambda i: (0,)),
            pl.BlockSpec((H,), lambda i: (0,)),
        ],
        out_specs=pl.BlockSpec((NB, H), lambda i: (i, 0)),
        out_shape=jax.ShapeDtypeStruct((N, H), jnp.float32),
    )(node_features, p['enc_W1'], p['enc_b1'], p['enc_W2'], p['enc_b2'],
      p['enc_ln_g'], p['enc_ln_b'])


def _gat_jnp(x, src, dst, Wl, asl, adl, bl, n):
    heads = []
    for hh in range(HEADS):
        h = x @ Wl[hh]
        asrc = (h * asl[hh]).sum(-1)
        adst = (h * adl[hh]).sum(-1)
        e = jax.nn.leaky_relu(asrc[src] + adst[dst], 0.2)
        m = jax.ops.segment_max(e, dst, num_segments=n)
        ex = jnp.exp(e - m[dst])
        s = jax.ops.segment_sum(ex, dst, num_segments=n)
        alpha = ex / (s[dst] + 1e-16)
        heads.append(jax.ops.segment_sum(alpha[:, None] * h[src], dst, num_segments=n) + bl[hh])
    return jnp.concatenate(heads, axis=-1)


def kernel(node_features, edge_index, params):
    p = params
    n = N
    loop = jnp.arange(n, dtype=edge_index.dtype)
    src = jnp.concatenate([edge_index[0], loop])
    dst = jnp.concatenate([edge_index[1], loop])
    deg = jax.ops.segment_sum(jnp.ones(src.shape[0], dtype=jnp.float32), dst, num_segments=n)
    dinv = jnp.where(deg > 0, 1.0 / jnp.sqrt(deg), 0.0)

    x = _encoder(node_features, p)

    attn = x
    for i in range(L):
        out = _gat_jnp(attn, src, dst, p['gat_W'][i], p['gat_asrc'][i],
                       p['gat_adst'][i], p['gat_b'][i], n)
        out = out @ p['proj_W'][i] + p['proj_b'][i]
        m = out.mean(-1, keepdims=True)
        v = ((out - m) ** 2).mean(-1, keepdims=True)
        out = (out - m) / jnp.sqrt(v + 1e-5) * p['ln_g'][i] + p['ln_b'][i]
        attn = attn + out

    trad = x
    for i in range(L):
        h = trad @ p['gcn_W'][i]
        norm = dinv[src] * dinv[dst]
        agg = jax.ops.segment_sum(norm[:, None] * h[src], dst, num_segments=n) + p['gcn_b'][i]
        trad = trad + jax.nn.relu(agg)

    combined = attn + trad
    g = jnp.concatenate([combined.mean(axis=0), combined.max(axis=0)])

    def mlp3(v, W1, b1, W2, b2, W3, b3):
        h1 = jax.nn.relu(v @ W1 + b1)
        h2 = jax.nn.relu(h1 @ W2 + b2)
        return h2 @ W3 + b3

    dec = jax.nn.sigmoid(mlp3(g, p['dec_W1'], p['dec_b1'], p['dec_W2'], p['dec_b2'], p['dec_W3'], p['dec_b3']))
    val = mlp3(g, p['val_W1'], p['val_b1'], p['val_W2'], p['val_b2'], p['val_W3'], p['val_b3'])
    temp = jax.nn.relu(g @ p['tmp_W1'] + p['tmp_b1']) @ p['tmp_W2'] + p['tmp_b2']
    safe = jax.nn.sigmoid(jax.nn.relu(g @ p['safe_W1'] + p['safe_b1']) @ p['safe_W2'] + p['safe_b2'])
    return dec, val, temp, safe


# speedup vs baseline: 1.0767x; 1.0481x over previous
import functools

import jax
import jax.numpy as jnp
from jax import lax
from jax.experimental import pallas as pl
from jax.experimental.pallas import tpu as pltpu
from jax.experimental.pallas import tpu_sc as plsc

N = 10000
E = 320000
DF = 128
H = 128
L = 4
HEADS = 4
HD = H // HEADS

NC = 2
NS = 16
LANES = 16
W = NC * NS

NE_TOT = E + N
CH = 128
CPW = -(-NE_TOT // (W * CH))
EPW = CPW * CH
NE_PAD = W * EPW
DUMMY = N
NR = 10240
RPT = NR // (NS * CH)

NB = 400

_mesh_cache = []


def _mesh():
    if not _mesh_cache:
        _mesh_cache.append(plsc.VectorSubcoreMesh(
            core_axis_name="c", subcore_axis_name="s",
            num_cores=NC, num_subcores=NS))
    return _mesh_cache[0]


def _zero_vmem_rows(rows):
    def zrow(i, _):
        for j in range(H // LANES):
            rows[i, pl.ds(j * LANES, LANES)] = jnp.zeros((LANES,), jnp.float32)
        return 0
    lax.fori_loop(0, CH, zrow, 0)


@functools.cache
def _deg_kernel():
  kern = functools.partial(
    pl.kernel,
    out_type=jax.ShapeDtypeStruct((NC, NR), jnp.float32),
    mesh=_mesh(),
    scratch_types=[
        pltpu.VMEM((CH,), jnp.int32),
        pltpu.VMEM((CH,), jnp.float32),
        pltpu.VMEM((CH,), jnp.float32),
        pltpu.VMEM_SHARED((NR,), jnp.float32),
    ],
  )

  @kern
  def _deg_sc(dst_hbm, out_hbm, didx, ones_v, zero_v, dacc):
    c = lax.axis_index("c")
    s = lax.axis_index("s")
    w = c * NS + s
    for j in range(CH // LANES):
        ones_v[pl.ds(j * LANES, LANES)] = jnp.ones((LANES,), jnp.float32)
        zero_v[pl.ds(j * LANES, LANES)] = jnp.zeros((LANES,), jnp.float32)
    for t in range(NR // (NS * CH)):
        pltpu.sync_copy(zero_v, dacc.at[pl.ds((s * RPT + t) * CH, CH)])
    plsc.subcore_barrier()

    def body(i, _):
        base = w * EPW + i * CH
        pltpu.sync_copy(dst_hbm.at[pl.ds(base, CH)], didx)
        pltpu.sync_copy(ones_v, dacc.at[didx], add=True)
        return 0
    lax.fori_loop(0, CPW, body, 0)
    plsc.subcore_barrier()
    for t in range(RPT):
        r = (s * RPT + t) * CH
        pltpu.sync_copy(dacc.at[pl.ds(r, CH)], out_hbm.at[c, pl.ds(r, CH)])

  return _deg_sc


@functools.cache
def _gcn_kernel():
  kern = functools.partial(
    pl.kernel,
    out_type=jax.ShapeDtypeStruct((NC, NR, H), jnp.float32),
    mesh=_mesh(),
    scratch_types=[
        pltpu.VMEM((CH,), jnp.int32),
        pltpu.VMEM((CH,), jnp.int32),
        pltpu.VMEM((CH, H), jnp.float32),
        pltpu.VMEM_SHARED((NR, H), jnp.float32),
        pltpu.SemaphoreType.DMA,
    ],
  )

  @kern
  def _gcn_sc(hp_hbm, src_hbm, dst_hbm, out_hbm, sidx, didx, rows, accum, sem):
    c = lax.axis_index("c")
    s = lax.axis_index("s")
    w = c * NS + s
    _zero_vmem_rows(rows)
    for t in range(RPT):
        pltpu.sync_copy(rows, accum.at[pl.ds((s * RPT + t) * CH, CH)])
    plsc.subcore_barrier()

    def body(i, _):
        base = w * EPW + i * CH
        pltpu.sync_copy(src_hbm.at[pl.ds(base, CH)], sidx)
        pltpu.sync_copy(dst_hbm.at[pl.ds(base, CH)], didx)
        pltpu.async_copy(hp_hbm.at[sidx], rows, sem).wait()
        pltpu.sync_copy(rows, accum.at[didx], add=True)
        return 0
    lax.fori_loop(0, CPW, body, 0)
    plsc.subcore_barrier()
    for t in range(RPT):
        r = (s * RPT + t) * CH
        pltpu.sync_copy(accum.at[pl.ds(r, CH)], out_hbm.at[c, pl.ds(r, CH)])

  return _gcn_sc


def _encoder_body(nf_ref, w1_ref, b1_ref, w2_ref, b2_ref, g_ref, bb_ref, o_ref):
    x = jnp.maximum(jnp.dot(nf_ref[...], w1_ref[...],
                            preferred_element_type=jnp.float32) + b1_ref[...], 0.0)
    x = jnp.dot(x, w2_ref[...], preferred_element_type=jnp.float32) + b2_ref[...]
    m = x.mean(-1, keepdims=True)
    v = ((x - m) ** 2).mean(-1, keepdims=True)
    o_ref[...] = (x - m) * lax.rsqrt(v + 1e-5) * g_ref[...] + bb_ref[...]


def _encoder(node_features, p):
    return pl.pallas_call(
        _encoder_body,
        grid=(N // NB,),
        in_specs=[
            pl.BlockSpec((NB, DF), lambda i: (i, 0)),
            pl.BlockSpec((DF, H), lambda i: (0, 0)),
            pl.BlockSpec((H,), lambda i: (0,)),
            pl.BlockSpec((H, H), lambda i: (0, 0)),
            pl.BlockSpec((H,), lambda i: (0,)),
            pl.BlockSpec((H,), lambda i: (0,)),
            pl.BlockSpec((H,), lambda i: (0,)),
        ],
        out_specs=pl.BlockSpec((NB, H), lambda i: (i, 0)),
        out_shape=jax.ShapeDtypeStruct((N, H), jnp.float32),
    )(node_features, p['enc_W1'], p['enc_b1'], p['enc_W2'], p['enc_b2'],
      p['enc_ln_g'], p['enc_ln_b'])


def _gat_jnp(x, src, dst, Wl, asl, adl, bl, n):
    heads = []
    for hh in range(HEADS):
        h = x @ Wl[hh]
        asrc = (h * asl[hh]).sum(-1)
        adst = (h * adl[hh]).sum(-1)
        e = jax.nn.leaky_relu(asrc[src] + adst[dst], 0.2)
        m = jax.ops.segment_max(e, dst, num_segments=n)
        ex = jnp.exp(e - m[dst])
        s = jax.ops.segment_sum(ex, dst, num_segments=n)
        alpha = ex / (s[dst] + 1e-16)
        heads.append(jax.ops.segment_sum(alpha[:, None] * h[src], dst, num_segments=n) + bl[hh])
    return jnp.concatenate(heads, axis=-1)


def kernel(node_features, edge_index, params):
    p = params
    n = N
    loop = jnp.arange(n, dtype=edge_index.dtype)
    src = jnp.concatenate([edge_index[0], loop])
    dst = jnp.concatenate([edge_index[1], loop])
    pad = jnp.full((NE_PAD - NE_TOT,), DUMMY, dtype=edge_index.dtype)
    srcw = jnp.concatenate([src, pad])
    dstw = jnp.concatenate([dst, pad])

    deg2 = _deg_kernel()(dstw)
    deg = (deg2[0] + deg2[1])[:n]
    dinv = jnp.where(deg > 0, 1.0 / jnp.sqrt(deg), 0.0)

    x = _encoder(node_features, p)

    attn = x
    for i in range(L):
        out = _gat_jnp(attn, src, dst, p['gat_W'][i], p['gat_asrc'][i],
                       p['gat_adst'][i], p['gat_b'][i], n)
        out = out @ p['proj_W'][i] + p['proj_b'][i]
        m = out.mean(-1, keepdims=True)
        v = ((out - m) ** 2).mean(-1, keepdims=True)
        out = (out - m) / jnp.sqrt(v + 1e-5) * p['ln_g'][i] + p['ln_b'][i]
        attn = attn + out

    trad = x
    for i in range(L):
        hp = jnp.zeros((NR, H), jnp.float32).at[:n].set(dinv[:, None] * (trad @ p['gcn_W'][i]))
        agg2 = _gcn_kernel()(hp, srcw, dstw)
        agg = dinv[:, None] * (agg2[0] + agg2[1])[:n] + p['gcn_b'][i]
        trad = trad + jax.nn.relu(agg)

    combined = attn + trad
    g = jnp.concatenate([combined.mean(axis=0), combined.max(axis=0)])

    def mlp3(v, W1, b1, W2, b2, W3, b3):
        h1 = jax.nn.relu(v @ W1 + b1)
        h2 = jax.nn.relu(h1 @ W2 + b2)
        return h2 @ W3 + b3

    dec = jax.nn.sigmoid(mlp3(g, p['dec_W1'], p['dec_b1'], p['dec_W2'], p['dec_b2'], p['dec_W3'], p['dec_b3']))
    val = mlp3(g, p['val_W1'], p['val_b1'], p['val_W2'], p['val_b2'], p['val_W3'], p['val_b3'])
    temp = jax.nn.relu(g @ p['tmp_W1'] + p['tmp_b1']) @ p['tmp_W2'] + p['tmp_b2']
    safe = jax.nn.sigmoid(jax.nn.relu(g @ p['safe_W1'] + p['safe_b1']) @ p['safe_W2'] + p['safe_b2'])
    return dec, val, temp, safe


# speedup vs baseline: 37.6104x; 34.9312x over previous
import functools

import jax
import jax.numpy as jnp
from jax import lax
from jax.experimental import pallas as pl
from jax.experimental.pallas import tpu as pltpu
from jax.experimental.pallas import tpu_sc as plsc

N = 10000
E = 320000
DF = 128
H = 128
L = 4
HEADS = 4
HD = H // HEADS

NC = 2
NS = 16
LANES = 16
W = NC * NS

NE_TOT = E + N
CH = 128
CPW = -(-NE_TOT // (W * CH))
EPW = CPW * CH
NE_PAD = W * EPW
DUMMY = N
NR = 10240
RPT = NR // (NS * CH)

NB = 400

_mesh_cache = []


def _mesh():
    if not _mesh_cache:
        _mesh_cache.append(plsc.VectorSubcoreMesh(
            core_axis_name="c", subcore_axis_name="s",
            num_cores=NC, num_subcores=NS))
    return _mesh_cache[0]


def _zero_vmem_rows(rows):
    def zrow(i, _):
        for j in range(H // LANES):
            rows[i, pl.ds(j * LANES, LANES)] = jnp.zeros((LANES,), jnp.float32)
        return 0
    lax.fori_loop(0, CH, zrow, 0)


@functools.cache
def _deg_kernel():
  kern = functools.partial(
    pl.kernel,
    out_type=jax.ShapeDtypeStruct((NC, NR), jnp.float32),
    mesh=_mesh(),
    scratch_types=[
        pltpu.VMEM((CH,), jnp.int32),
        pltpu.VMEM((CH,), jnp.float32),
        pltpu.VMEM((CH,), jnp.float32),
        pltpu.VMEM_SHARED((NR,), jnp.float32),
    ],
  )

  @kern
  def _deg_sc(dst_hbm, out_hbm, didx, ones_v, zero_v, dacc):
    c = lax.axis_index("c")
    s = lax.axis_index("s")
    w = c * NS + s
    for j in range(CH // LANES):
        ones_v[pl.ds(j * LANES, LANES)] = jnp.ones((LANES,), jnp.float32)
        zero_v[pl.ds(j * LANES, LANES)] = jnp.zeros((LANES,), jnp.float32)
    for t in range(NR // (NS * CH)):
        pltpu.sync_copy(zero_v, dacc.at[pl.ds((s * RPT + t) * CH, CH)])
    plsc.subcore_barrier()

    def body(i, _):
        base = w * EPW + i * CH
        pltpu.sync_copy(dst_hbm.at[pl.ds(base, CH)], didx)
        pltpu.sync_copy(ones_v, dacc.at[didx], add=True)
        return 0
    lax.fori_loop(0, CPW, body, 0)
    plsc.subcore_barrier()
    for t in range(RPT):
        r = (s * RPT + t) * CH
        pltpu.sync_copy(dacc.at[pl.ds(r, CH)], out_hbm.at[c, pl.ds(r, CH)])

  return _deg_sc


@functools.cache
def _gcn_kernel():
  kern = functools.partial(
    pl.kernel,
    out_type=jax.ShapeDtypeStruct((NC, NR, H), jnp.float32),
    mesh=_mesh(),
    scratch_types=[
        pltpu.VMEM((CH,), jnp.int32),
        pltpu.VMEM((CH,), jnp.int32),
        pltpu.VMEM((CH, H), jnp.float32),
        pltpu.VMEM_SHARED((NR, H), jnp.float32),
        pltpu.SemaphoreType.DMA,
    ],
  )

  @kern
  def _gcn_sc(hp_hbm, src_hbm, dst_hbm, out_hbm, sidx, didx, rows, accum, sem):
    c = lax.axis_index("c")
    s = lax.axis_index("s")
    w = c * NS + s
    _zero_vmem_rows(rows)
    for t in range(RPT):
        pltpu.sync_copy(rows, accum.at[pl.ds((s * RPT + t) * CH, CH)])
    plsc.subcore_barrier()

    def body(i, _):
        base = w * EPW + i * CH
        pltpu.sync_copy(src_hbm.at[pl.ds(base, CH)], sidx)
        pltpu.sync_copy(dst_hbm.at[pl.ds(base, CH)], didx)
        pltpu.async_copy(hp_hbm.at[sidx], rows, sem).wait()
        pltpu.sync_copy(rows, accum.at[didx], add=True)
        return 0
    lax.fori_loop(0, CPW, body, 0)
    plsc.subcore_barrier()
    for t in range(RPT):
        r = (s * RPT + t) * CH
        pltpu.sync_copy(accum.at[pl.ds(r, CH)], out_hbm.at[c, pl.ds(r, CH)])

  return _gcn_sc


SPT = NR * HEADS // (NS * CH)


@functools.cache
def _att_kernel():
  kern = functools.partial(
    pl.kernel,
    out_type=(jax.ShapeDtypeStruct((HEADS, NE_PAD), jnp.float32),
              jax.ShapeDtypeStruct((NC, NR * HEADS), jnp.float32)),
    mesh=_mesh(),
    compiler_params=pltpu.CompilerParams(needs_layout_passes=False),
    scratch_types=[
        pltpu.VMEM((NR * HEADS,), jnp.float32),
        pltpu.VMEM((NR * HEADS,), jnp.float32),
        pltpu.VMEM((LANES,), jnp.float32),
        pltpu.VMEM((CH,), jnp.int32),
        pltpu.VMEM((CH,), jnp.int32),
        pltpu.VMEM((HEADS, CH), jnp.float32),
        pltpu.VMEM((HEADS, CH), jnp.int32),
        pltpu.VMEM((CH,), jnp.float32),
        pltpu.VMEM_SHARED((NR * HEADS,), jnp.float32),
    ],
  )

  @kern
  def _att_sc(as_hbm, ad_hbm, m_hbm, src_hbm, dst_hbm, ex_hbm, s_hbm,
              as_v, ad_v, m_v, sidx, didx, exb, sxb, zbuf, sacc):
    c = lax.axis_index("c")
    s = lax.axis_index("s")
    w = c * NS + s
    pltpu.sync_copy(as_hbm, as_v)
    pltpu.sync_copy(ad_hbm, ad_v)
    pltpu.sync_copy(m_hbm, m_v)
    mvec = m_v[...]
    for j in range(CH // LANES):
        zbuf[pl.ds(j * LANES, LANES)] = jnp.zeros((LANES,), jnp.float32)
    for t in range(SPT):
        pltpu.sync_copy(zbuf, sacc.at[pl.ds((s * SPT + t) * CH, CH)])
    plsc.subcore_barrier()

    def body(i, _):
        base = w * EPW + i * CH
        pltpu.sync_copy(src_hbm.at[pl.ds(base, CH)], sidx)
        pltpu.sync_copy(dst_hbm.at[pl.ds(base, CH)], didx)
        for g in range(CH // LANES):
            sv = sidx[pl.ds(g * LANES, LANES)] * HEADS
            dv = didx[pl.ds(g * LANES, LANES)] * HEADS
            for h in range(HEADS):
                av = plsc.load_gather(as_v, [sv + h])
                bv = plsc.load_gather(ad_v, [dv + h])
                z = av + bv
                e = jnp.where(z >= 0, z, z * 0.2) - mvec[h]
                exb[h, pl.ds(g * LANES, LANES)] = jnp.exp(e)
                sxb[h, pl.ds(g * LANES, LANES)] = dv + h
        for h in range(HEADS):
            pltpu.sync_copy(exb.at[h], ex_hbm.at[h, pl.ds(base, CH)])
            pltpu.sync_copy(exb.at[h], sacc.at[sxb.at[h]], add=True)
        return 0
    lax.fori_loop(0, CPW, body, 0)
    plsc.subcore_barrier()
    for t in range(SPT):
        r = (s * SPT + t) * CH
        pltpu.sync_copy(sacc.at[pl.ds(r, CH)], s_hbm.at[c, pl.ds(r, CH)])

  return _att_sc


@functools.cache
def _msg_kernel():
  kern = functools.partial(
    pl.kernel,
    out_type=jax.ShapeDtypeStruct((NC, NR, H), jnp.float32),
    mesh=_mesh(),
    scratch_types=[
        pltpu.VMEM((CH,), jnp.int32),
        pltpu.VMEM((CH,), jnp.int32),
        pltpu.VMEM((CH, H), jnp.float32),
        pltpu.VMEM((HEADS, CH), jnp.float32),
        pltpu.VMEM_SHARED((NR, H), jnp.float32),
        pltpu.SemaphoreType.DMA,
    ],
  )

  @kern
  def _msg_sc(hp_hbm, src_hbm, dst_hbm, ex_hbm, out_hbm,
              sidx, didx, rows, exb, accum, sem):
    c = lax.axis_index("c")
    s = lax.axis_index("s")
    w = c * NS + s
    _zero_vmem_rows(rows)
    for t in range(RPT):
        pltpu.sync_copy(rows, accum.at[pl.ds((s * RPT + t) * CH, CH)])
    plsc.subcore_barrier()

    def body(i, _):
        base = w * EPW + i * CH
        pltpu.sync_copy(src_hbm.at[pl.ds(base, CH)], sidx)
        pltpu.sync_copy(dst_hbm.at[pl.ds(base, CH)], didx)
        for h in range(HEADS):
            pltpu.sync_copy(ex_hbm.at[h, pl.ds(base, CH)], exb.at[h])
        pltpu.async_copy(hp_hbm.at[sidx], rows, sem).wait()

        def mul(g, _):
            exv = [exb[h, pl.ds(g * LANES, LANES)] for h in range(HEADS)]
            for el in range(LANES):
                e2 = g * LANES + el
                for h in range(HEADS):
                    x = exv[h][el]
                    for k2 in range(HD // LANES):
                        off = h * HD + k2 * LANES
                        rows[e2, pl.ds(off, LANES)] = rows[e2, pl.ds(off, LANES)] * x
            return 0
        lax.fori_loop(0, CH // LANES, mul, 0)
        pltpu.sync_copy(rows, accum.at[didx], add=True)
        return 0
    lax.fori_loop(0, CPW, body, 0)
    plsc.subcore_barrier()
    for t in range(RPT):
        r = (s * RPT + t) * CH
        pltpu.sync_copy(accum.at[pl.ds(r, CH)], out_hbm.at[c, pl.ds(r, CH)])

  return _msg_sc


def _gat_sc(attn, srcw, dstw, Wl, asl, adl, bl):
    Wcat = jnp.moveaxis(Wl, 0, 1).reshape(H, H)
    h = attn @ Wcat
    hh = h.reshape(N, HEADS, HD)
    asn = (hh * asl[None]).sum(-1)
    adn = (hh * adl[None]).sum(-1)
    M = jnp.max(asn, axis=0) + jnp.max(adn, axis=0)
    M = jnp.where(M >= 0, M, 0.2 * M)
    Mp = jnp.zeros((LANES,), jnp.float32).at[:HEADS].set(M)
    asp = jnp.zeros((NR, HEADS), jnp.float32).at[:N].set(asn).reshape(-1)
    adp = jnp.zeros((NR, HEADS), jnp.float32).at[:N].set(adn).reshape(-1)
    ex, s2 = _att_kernel()(asp, adp, Mp, srcw, dstw)
    sn = (s2[0] + s2[1]).reshape(NR, HEADS)[:N]
    hp = jnp.zeros((NR, H), jnp.float32).at[:N].set(h)
    agg2 = _msg_kernel()(hp, srcw, dstw, ex)
    agg = (agg2[0] + agg2[1])[:N].reshape(N, HEADS, HD)
    out = agg / (sn[:, :, None] + 1e-16) + bl[None]
    return out.reshape(N, H)


def _encoder_body(nf_ref, w1_ref, b1_ref, w2_ref, b2_ref, g_ref, bb_ref, o_ref):
    x = jnp.maximum(jnp.dot(nf_ref[...], w1_ref[...],
                            preferred_element_type=jnp.float32) + b1_ref[...], 0.0)
    x = jnp.dot(x, w2_ref[...], preferred_element_type=jnp.float32) + b2_ref[...]
    m = x.mean(-1, keepdims=True)
    v = ((x - m) ** 2).mean(-1, keepdims=True)
    o_ref[...] = (x - m) * lax.rsqrt(v + 1e-5) * g_ref[...] + bb_ref[...]


def _encoder(node_features, p):
    return pl.pallas_call(
        _encoder_body,
        grid=(N // NB,),
        in_specs=[
            pl.BlockSpec((NB, DF), lambda i: (i, 0)),
            pl.BlockSpec((DF, H), lambda i: (0, 0)),
            pl.BlockSpec((H,), lambda i: (0,)),
            pl.BlockSpec((H, H), lambda i: (0, 0)),
            pl.BlockSpec((H,), lambda i: (0,)),
            pl.BlockSpec((H,), lambda i: (0,)),
            pl.BlockSpec((H,), lambda i: (0,)),
        ],
        out_specs=pl.BlockSpec((NB, H), lambda i: (i, 0)),
        out_shape=jax.ShapeDtypeStruct((N, H), jnp.float32),
    )(node_features, p['enc_W1'], p['enc_b1'], p['enc_W2'], p['enc_b2'],
      p['enc_ln_g'], p['enc_ln_b'])


def _gat_jnp(x, src, dst, Wl, asl, adl, bl, n):
    heads = []
    for hh in range(HEADS):
        h = x @ Wl[hh]
        asrc = (h * asl[hh]).sum(-1)
        adst = (h * adl[hh]).sum(-1)
        e = jax.nn.leaky_relu(asrc[src] + adst[dst], 0.2)
        m = jax.ops.segment_max(e, dst, num_segments=n)
        ex = jnp.exp(e - m[dst])
        s = jax.ops.segment_sum(ex, dst, num_segments=n)
        alpha = ex / (s[dst] + 1e-16)
        heads.append(jax.ops.segment_sum(alpha[:, None] * h[src], dst, num_segments=n) + bl[hh])
    return jnp.concatenate(heads, axis=-1)


def kernel(node_features, edge_index, params):
    p = params
    n = N
    loop = jnp.arange(n, dtype=edge_index.dtype)
    src = jnp.concatenate([edge_index[0], loop])
    dst = jnp.concatenate([edge_index[1], loop])
    pad = jnp.full((NE_PAD - NE_TOT,), DUMMY, dtype=edge_index.dtype)
    srcw = jnp.concatenate([src, pad])
    dstw = jnp.concatenate([dst, pad])

    deg2 = _deg_kernel()(dstw)
    deg = (deg2[0] + deg2[1])[:n]
    dinv = jnp.where(deg > 0, 1.0 / jnp.sqrt(deg), 0.0)

    x = _encoder(node_features, p)

    attn = x
    for i in range(L):
        out = _gat_sc(attn, srcw, dstw, p['gat_W'][i], p['gat_asrc'][i],
                      p['gat_adst'][i], p['gat_b'][i])
        out = out @ p['proj_W'][i] + p['proj_b'][i]
        m = out.mean(-1, keepdims=True)
        v = ((out - m) ** 2).mean(-1, keepdims=True)
        out = (out - m) / jnp.sqrt(v + 1e-5) * p['ln_g'][i] + p['ln_b'][i]
        attn = attn + out

    trad = x
    for i in range(L):
        hp = jnp.zeros((NR, H), jnp.float32).at[:n].set(dinv[:, None] * (trad @ p['gcn_W'][i]))
        agg2 = _gcn_kernel()(hp, srcw, dstw)
        agg = dinv[:, None] * (agg2[0] + agg2[1])[:n] + p['gcn_b'][i]
        trad = trad + jax.nn.relu(agg)

    combined = attn + trad
    g = jnp.concatenate([combined.mean(axis=0), combined.max(axis=0)])

    def mlp3(v, W1, b1, W2, b2, W3, b3):
        h1 = jax.nn.relu(v @ W1 + b1)
        h2 = jax.nn.relu(h1 @ W2 + b2)
        return h2 @ W3 + b3

    dec = jax.nn.sigmoid(mlp3(g, p['dec_W1'], p['dec_b1'], p['dec_W2'], p['dec_b2'], p['dec_W3'], p['dec_b3']))
    val = mlp3(g, p['val_W1'], p['val_b1'], p['val_W2'], p['val_b2'], p['val_W3'], p['val_b3'])
    temp = jax.nn.relu(g @ p['tmp_W1'] + p['tmp_b1']) @ p['tmp_W2'] + p['tmp_b2']
    safe = jax.nn.sigmoid(jax.nn.relu(g @ p['safe_W1'] + p['safe_b1']) @ p['safe_W2'] + p['safe_b2'])
    return dec, val, temp, safe
